# baseline (speedup 1.0000x reference)
import functools

import jax
import jax.numpy as jnp
from jax import lax
from jax.experimental import pallas as pl
from jax.experimental.pallas import tpu as pltpu
from jax.experimental.pallas import tpu_sc as plsc

VOCAB = 100000
EMB = 300
FFN = 300
B, L = 4096, 200
BL = B * L

_FFN_PAD = 512
_PACK = _FFN_PAD // 2
_HI = FFN - _PACK

_NC = 2
_NS = 16
_NW = _NC * _NS
_CHUNK = 128


_ROWS_BLK = 4096


def _bf16_bits_hi(x):
    return lax.bitcast_convert_type(
        x.astype(jnp.bfloat16).astype(jnp.float32), jnp.uint32
    ) & jnp.uint32(0xFFFF0000)


def _proj_body(tblT_ref, w_ref, out_ref):
    acc = lax.dot_general(
        tblT_ref[...], w_ref[...],
        dimension_numbers=(((0,), (1,)), ((), ())),
        preferred_element_type=jnp.float32,
    )
    lo = _bf16_bits_hi(acc[:, :_PACK]) >> 16
    hi = _bf16_bits_hi(acc[:, _PACK:])
    out_ref[...] = lax.bitcast_convert_type(lo | hi, jnp.int32)


def _project(tableT, Wpad):
    return pl.pallas_call(
        _proj_body,
        grid=(pl.cdiv(VOCAB, _ROWS_BLK),),
        in_specs=[
            pl.BlockSpec((EMB, _ROWS_BLK), lambda i: (0, i)),
            pl.BlockSpec((_FFN_PAD, EMB), lambda i: (0, 0)),
        ],
        out_specs=pl.BlockSpec((_ROWS_BLK, _PACK), lambda i: (i, 0)),
        out_shape=jax.ShapeDtypeStruct((VOCAB, _PACK), jnp.int32),
    )(tableT, Wpad)


_SLABS = (8, 16, 32, 48, 48, 32, 16)


def _gather_slab(P, idx_s, nchs):
    mesh = plsc.VectorSubcoreMesh(core_axis_name="c", subcore_axis_name="s")
    per_w = nchs * _CHUNK

    @functools.partial(
        pl.kernel,
        out_type=jax.ShapeDtypeStruct((per_w * _NW, _PACK), jnp.int32),
        mesh=mesh,
        scratch_types=[
            pltpu.VMEM((nchs, _CHUNK), jnp.int32),
            pltpu.VMEM((_CHUNK, _PACK), jnp.int32),
            pltpu.VMEM((_CHUNK, _PACK), jnp.int32),
            pltpu.SemaphoreType.DMA,
            pltpu.SemaphoreType.DMA,
        ],
    )
    def k(p_hbm, idx_hbm, out_hbm, idx_v, buf0, buf1, sem0, sem1):
        wid = lax.axis_index("s") * _NC + lax.axis_index("c")
        base = wid * per_w
        pltpu.sync_copy(idx_hbm.at[wid], idx_v)

        bufs = (buf0, buf1)
        sems = (sem0, sem1)

        pltpu.async_copy(p_hbm.at[idx_v.at[0]], buf0, sem0)
        pltpu.async_copy(p_hbm.at[idx_v.at[1]], buf1, sem1)

        def step2(jj, carry):
            j = jj * 2
            for par in range(2):
                buf, sem = bufs[par], sems[par]
                pltpu.make_async_copy(p_hbm.at[idx_v.at[j + par]], buf, sem).wait()
                pltpu.sync_copy(buf, out_hbm.at[pl.ds(base + (j + par) * _CHUNK, _CHUNK)])

                @pl.when(j + par + 2 < nchs)
                def _():
                    pltpu.async_copy(p_hbm.at[idx_v.at[j + par + 2]], buf, sem)
            return carry

        lax.fori_loop(0, nchs // 2, step2, 0)

    return k(P, idx_s)


_LBLK = 8
_BBLK = 512


def _unpackT_body(g_ref, out_ref):
    t = jnp.transpose(g_ref[...], (2, 0, 1))
    u = lax.bitcast_convert_type(t, jnp.uint32)
    low = lax.bitcast_convert_type(u << 16, jnp.float32)
    high = lax.bitcast_convert_type(u[:_HI] & jnp.uint32(0xFFFF0000), jnp.float32)
    out_ref[...] = jnp.concatenate([low, high], axis=0)


def _unpack_slab(G3, carry, l0, ls):
    def body(g_ref, *refs):
        _unpackT_body(g_ref, refs[-1])

    nlg = ls // _LBLK
    lg0 = l0 // _LBLK
    in_specs = [pl.BlockSpec((_LBLK, _BBLK, _PACK), lambda li, bi: (li, bi, 0))]
    args = [G3]
    aliases = {}
    if carry is not None:
        in_specs.append(pl.BlockSpec(memory_space=pltpu.MemorySpace.HBM))
        args.append(carry)
        aliases = {1: 0}
    return pl.pallas_call(
        body,
        grid=(nlg, B // _BBLK),
        in_specs=in_specs,
        out_specs=pl.BlockSpec(
            (FFN, _LBLK, _BBLK), lambda li, bi, lg0=lg0: (0, lg0 + li, bi)
        ),
        out_shape=jax.ShapeDtypeStruct((FFN, L, B), jnp.float32),
        input_output_aliases=aliases,
    )(*args)


def kernel(data, table, W):
    Wpad = jnp.pad(W, ((0, _FFN_PAD - FFN), (0, 0)))
    tableT = jnp.transpose(table)
    P = _project(tableT, Wpad)
    dataT = jnp.transpose(data)
    out = None
    l0 = 0
    for ls in _SLABS:
        idx_s = jnp.reshape(lax.slice_in_dim(dataT, l0, l0 + ls), (_NW, ls, _CHUNK))
        G = _gather_slab(P, idx_s, ls)
        G3 = jnp.reshape(G, (ls, B, _PACK))
        out = _unpack_slab(G3, out, l0, ls)
        l0 += ls
    return jnp.transpose(out, (2, 1, 0))


# speedup vs baseline: 1.0008x; 1.0008x over previous
import functools

import jax
import jax.numpy as jnp
from jax import lax
from jax.experimental import pallas as pl
from jax.experimental.pallas import tpu as pltpu
from jax.experimental.pallas import tpu_sc as plsc

VOCAB = 100000
EMB = 300
FFN = 300
B, L = 4096, 200
BL = B * L

_FFN_PAD = 512
_PACK = _FFN_PAD // 2
_HI = FFN - _PACK

_NC = 2
_NS = 16
_NW = _NC * _NS
_CHUNK = 128


_ROWS_BLK = 4096


def _bf16_bits_hi(x):
    return lax.bitcast_convert_type(
        x.astype(jnp.bfloat16).astype(jnp.float32), jnp.uint32
    ) & jnp.uint32(0xFFFF0000)


def _proj_body(tblT_ref, w_ref, out_ref):
    acc = lax.dot_general(
        tblT_ref[...], w_ref[...],
        dimension_numbers=(((0,), (1,)), ((), ())),
        preferred_element_type=jnp.float32,
    )
    lo = _bf16_bits_hi(acc[:, :_PACK]) >> 16
    hi = _bf16_bits_hi(acc[:, _PACK:])
    out_ref[...] = lax.bitcast_convert_type(lo | hi, jnp.int32)


def _project(tableT, Wpad):
    return pl.pallas_call(
        _proj_body,
        grid=(pl.cdiv(VOCAB, _ROWS_BLK),),
        in_specs=[
            pl.BlockSpec((EMB, _ROWS_BLK), lambda i: (0, i)),
            pl.BlockSpec((_FFN_PAD, EMB), lambda i: (0, 0)),
        ],
        out_specs=pl.BlockSpec((_ROWS_BLK, _PACK), lambda i: (i, 0)),
        out_shape=jax.ShapeDtypeStruct((VOCAB, _PACK), jnp.int32),
    )(tableT, Wpad)


_SLABS = (8, 24, 48, 56, 48, 16)


def _gather_slab(P, idx_s, nchs):
    mesh = plsc.VectorSubcoreMesh(core_axis_name="c", subcore_axis_name="s")
    per_w = nchs * _CHUNK

    @functools.partial(
        pl.kernel,
        out_type=jax.ShapeDtypeStruct((per_w * _NW, _PACK), jnp.int32),
        mesh=mesh,
        scratch_types=[
            pltpu.VMEM((nchs, _CHUNK), jnp.int32),
            pltpu.VMEM((_CHUNK, _PACK), jnp.int32),
            pltpu.VMEM((_CHUNK, _PACK), jnp.int32),
            pltpu.SemaphoreType.DMA,
            pltpu.SemaphoreType.DMA,
        ],
    )
    def k(p_hbm, idx_hbm, out_hbm, idx_v, buf0, buf1, sem0, sem1):
        wid = lax.axis_index("s") * _NC + lax.axis_index("c")
        base = wid * per_w
        pltpu.sync_copy(idx_hbm.at[wid], idx_v)

        bufs = (buf0, buf1)
        sems = (sem0, sem1)

        pltpu.async_copy(p_hbm.at[idx_v.at[0]], buf0, sem0)
        pltpu.async_copy(p_hbm.at[idx_v.at[1]], buf1, sem1)

        def step2(jj, carry):
            j = jj * 2
            for par in range(2):
                buf, sem = bufs[par], sems[par]
                pltpu.make_async_copy(p_hbm.at[idx_v.at[j + par]], buf, sem).wait()
                pltpu.sync_copy(buf, out_hbm.at[pl.ds(base + (j + par) * _CHUNK, _CHUNK)])

                @pl.when(j + par + 2 < nchs)
                def _():
                    pltpu.async_copy(p_hbm.at[idx_v.at[j + par + 2]], buf, sem)
            return carry

        lax.fori_loop(0, nchs // 2, step2, 0)

    return k(P, idx_s)


_LBLK = 8
_BBLK = 512


def _unpackT_body(g_ref, out_ref):
    t = jnp.transpose(g_ref[...], (2, 0, 1))
    u = lax.bitcast_convert_type(t, jnp.uint32)
    low = lax.bitcast_convert_type(u << 16, jnp.float32)
    high = lax.bitcast_convert_type(u[:_HI] & jnp.uint32(0xFFFF0000), jnp.float32)
    out_ref[...] = jnp.concatenate([low, high], axis=0)


def _unpack_slab(G3, carry, l0, ls):
    def body(g_ref, *refs):
        _unpackT_body(g_ref, refs[-1])

    nlg = ls // _LBLK
    lg0 = l0 // _LBLK
    in_specs = [pl.BlockSpec((_LBLK, _BBLK, _PACK), lambda li, bi: (li, bi, 0))]
    args = [G3]
    aliases = {}
    if carry is not None:
        in_specs.append(pl.BlockSpec(memory_space=pltpu.MemorySpace.HBM))
        args.append(carry)
        aliases = {1: 0}
    return pl.pallas_call(
        body,
        grid=(nlg, B // _BBLK),
        in_specs=in_specs,
        out_specs=pl.BlockSpec(
            (FFN, _LBLK, _BBLK), lambda li, bi, lg0=lg0: (0, lg0 + li, bi)
        ),
        out_shape=jax.ShapeDtypeStruct((FFN, L, B), jnp.float32),
        input_output_aliases=aliases,
    )(*args)


def kernel(data, table, W):
    Wpad = jnp.pad(W, ((0, _FFN_PAD - FFN), (0, 0)))
    tableT = jnp.transpose(table)
    P = _project(tableT, Wpad)
    dataT = jnp.transpose(data)
    out = None
    l0 = 0
    for ls in _SLABS:
        idx_s = jnp.reshape(lax.slice_in_dim(dataT, l0, l0 + ls), (_NW, ls, _CHUNK))
        G = _gather_slab(P, idx_s, ls)
        G3 = jnp.reshape(G, (ls, B, _PACK))
        out = _unpack_slab(G3, out, l0, ls)
        l0 += ls
    return jnp.transpose(out, (2, 1, 0))
